# baseline (speedup 1.0000x reference)
import functools

import jax
import jax.numpy as jnp
from jax.experimental import pallas as pl
from jax.experimental.pallas import tpu as pltpu

END_TOK = 49407
BLK = 512


def _body(plen_ref, vt_ref, end_ref, out_ref):
    b = pl.program_id(0)
    j = pl.program_id(1)
    plen = plen_ref[b]
    base = j * BLK
    rows = jax.lax.broadcasted_iota(jnp.int32, (1, BLK, 1), 1) + base
    mask = rows < plen
    end = end_ref[END_TOK % 8, :].reshape(1, 1, end_ref.shape[-1])
    out_ref[...] = jnp.where(mask, vt_ref[...], end)


def kernel(categories, vt, emb):
    B, L = categories.shape
    D = vt.shape[-1]
    NJ = pl.cdiv(L + 1, BLK)
    last_vt_blk = L // BLK - 1

    plen = jnp.sum((categories != END_TOK).astype(jnp.int32), axis=1)

    def vt_map(b, j, plen_ref):
        jc = jnp.minimum(j, (plen_ref[b] - 1) // BLK)
        return b, jnp.clip(jc, 0, last_vt_blk), 0

    def end_map(b, j, plen_ref):
        return END_TOK // 8, 0

    grid_spec = pltpu.PrefetchScalarGridSpec(
        num_scalar_prefetch=1,
        grid=(B, NJ),
        in_specs=[
            pl.BlockSpec((1, BLK, D), vt_map),
            pl.BlockSpec((8, D), end_map),
        ],
        out_specs=pl.BlockSpec((1, BLK, D), lambda b, j, p: (b, j, 0)),
    )

    return pl.pallas_call(
        _body,
        grid_spec=grid_spec,
        out_shape=jax.ShapeDtypeStruct((B, L + 1, D), vt.dtype),
        compiler_params=pltpu.CompilerParams(
            dimension_semantics=("arbitrary", "arbitrary"),
        ),
    )(plen, vt, emb)
